# baseline (speedup 1.0000x reference)
import functools

import jax
import jax.numpy as jnp
from jax.experimental import pallas as pl
from jax.experimental.pallas import tpu as pltpu


def _pool_kernel(x_ref, o_ref):
    o_ref[...] = jnp.sum(x_ref[...], axis=2, keepdims=True)


def _linear_kernel(p_ref, w_ref, b_ref, o_ref, *, inv_hw):
    acc = jax.lax.dot_general(
        p_ref[...],
        w_ref[...],
        dimension_numbers=(((1,), (1,)), ((), ())),
        preferred_element_type=jnp.float32,
    )
    o_ref[...] = acc * inv_hw + b_ref[...]


def _largest_divisor_at_most(n, cap):
    for d in range(min(n, cap), 0, -1):
        if n % d == 0:
            return d
    return 1


def kernel(x_nchw, weight, bias):
    B, C, H, W = x_nchw.shape
    N = weight.shape[0]
    HW = H * W
    out_dtype = jnp.result_type(x_nchw.dtype, weight.dtype)

    x = x_nchw.reshape(B, C, HW)

    bt = _largest_divisor_at_most(B, 8)
    grid = (B // bt,)

    pool_cost = pl.CostEstimate(
        flops=B * C * HW,
        transcendentals=0,
        bytes_accessed=x.size * x.dtype.itemsize + B * C * 4,
    )

    pooled3 = pl.pallas_call(
        _pool_kernel,
        out_shape=jax.ShapeDtypeStruct((B, C, 1), jnp.float32),
        grid=grid,
        in_specs=[pl.BlockSpec((bt, C, HW), lambda i: (i, 0, 0))],
        out_specs=pl.BlockSpec((bt, C, 1), lambda i: (i, 0, 0)),
        compiler_params=pltpu.CompilerParams(
            dimension_semantics=("parallel",),
            vmem_limit_bytes=56 << 20,
        ),
        cost_estimate=pool_cost,
    )(x)

    pooled = pooled3.reshape(B, C)
    bias2 = bias.reshape(1, N)

    bt2 = _largest_divisor_at_most(B, max(1, B // 2))
    lin_grid = (B // bt2,)

    lin_cost = pl.CostEstimate(
        flops=2 * B * C * N,
        transcendentals=0,
        bytes_accessed=B * C * 4 + N * C * weight.dtype.itemsize + B * N * 4,
    )

    scores = pl.pallas_call(
        functools.partial(_linear_kernel, inv_hw=float(1.0 / HW)),
        out_shape=jax.ShapeDtypeStruct((B, N), jnp.float32),
        grid=lin_grid,
        in_specs=[
            pl.BlockSpec((bt2, C), lambda i: (i, 0)),
            pl.BlockSpec((N, C), lambda i: (0, 0)),
            pl.BlockSpec((1, N), lambda i: (0, 0)),
        ],
        out_specs=pl.BlockSpec((bt2, N), lambda i: (i, 0)),
        compiler_params=pltpu.CompilerParams(
            dimension_semantics=("parallel",),
            vmem_limit_bytes=48 << 20,
        ),
        cost_estimate=lin_cost,
    )(pooled, weight, bias2)

    return scores.astype(out_dtype)
